# baseline (speedup 1.0000x reference)
import functools

import jax
import jax.numpy as jnp
from jax.experimental import pallas as pl

MASK_RATIO_ = 0.75
JCHUNK = 256
KTILE = 256


def _rank_kernel(ncol_ref, nrow_ref, rank_ref, mask_ref, *, L, len_keep):
    j = pl.program_id(1)
    a_j = ncol_ref[0, :, :]
    a_i = nrow_ref[0, :, :]
    lt = a_j < a_i
    eq = a_j == a_i
    jcol = jax.lax.broadcasted_iota(jnp.int32, (JCHUNK, 1), 0) + j * JCHUNK
    irow = jax.lax.broadcasted_iota(jnp.int32, (1, L), 1)
    contrib = lt | (eq & (jcol < irow))
    partial = jnp.sum(contrib.astype(jnp.int32), axis=0, keepdims=True)

    @pl.when(j == 0)
    def _():
        rank_ref[0, :, :] = partial

    @pl.when(j > 0)
    def _():
        rank_ref[0, :, :] += partial

    @pl.when(j == pl.num_programs(1) - 1)
    def _():
        mask_ref[0, :, :] = (rank_ref[0, :, :] >= len_keep).astype(jnp.float32)


def _ids_kernel(rank_ref, out_ref, *, L, E, koff):
    k = pl.program_id(1)
    rank_row = rank_ref[0, :, :]
    kcol = jax.lax.broadcasted_iota(jnp.int32, (KTILE, 1), 0) + koff + k * KTILE
    irow = jax.lax.broadcasted_iota(jnp.int32, (1, L), 1)
    eq = rank_row == kcol
    shuffle = jnp.sum(jnp.where(eq, irow, 0), axis=1, keepdims=True)
    out_ref[0, :, :] = jnp.broadcast_to(shuffle, (KTILE, E))


def kernel(x):
    B, L, E = x.shape
    len_keep = int(L * (1.0 - MASK_RATIO_))
    noise = jax.random.uniform(
        jax.random.fold_in(jax.random.key(0), 1), (B, L), dtype=jnp.float32)
    ncol = noise.reshape(B, L, 1)
    nrow = noise.reshape(B, 1, L)

    nj = L // JCHUNK
    rank, mask = pl.pallas_call(
        functools.partial(_rank_kernel, L=L, len_keep=len_keep),
        grid=(B, nj),
        in_specs=[
            pl.BlockSpec((1, JCHUNK, 1), lambda b, j: (b, j, 0)),
            pl.BlockSpec((1, 1, L), lambda b, j: (b, 0, 0)),
        ],
        out_specs=[
            pl.BlockSpec((1, 1, L), lambda b, j: (b, 0, 0)),
            pl.BlockSpec((1, 1, L), lambda b, j: (b, 0, 0)),
        ],
        out_shape=[
            jax.ShapeDtypeStruct((B, 1, L), jnp.int32),
            jax.ShapeDtypeStruct((B, 1, L), jnp.float32),
        ],
    )(ncol, nrow)

    def ids_call(koff, nrows):
        return pl.pallas_call(
            functools.partial(_ids_kernel, L=L, E=E, koff=koff),
            grid=(B, nrows // KTILE),
            in_specs=[pl.BlockSpec((1, 1, L), lambda b, k: (b, 0, 0))],
            out_specs=pl.BlockSpec((1, KTILE, E), lambda b, k: (b, k, 0)),
            out_shape=jax.ShapeDtypeStruct((B, nrows, E), jnp.int32),
        )(rank)

    ids_keep = ids_call(0, len_keep)
    ids_mask = ids_call(len_keep, L - len_keep)
    ids_restore = rank.reshape(B, L)
    return (mask.reshape(B, L), ids_keep, ids_restore, ids_mask)


# speedup vs baseline: 5.3384x; 5.3384x over previous
import functools

import jax
import jax.numpy as jnp
from jax.experimental import pallas as pl
from jax.experimental.pallas import tpu as pltpu

MASK_RATIO_ = 0.75
R, C = 64, 128
KTILE = 2048


def _xor_shuffle(x, d):
    if d < C:
        bit = jax.lax.broadcasted_iota(jnp.int32, (R, C), 1) & d
        return jnp.where(bit != 0, pltpu.roll(x, d, 1), pltpu.roll(x, C - d, 1))
    s = d // C
    bit = jax.lax.broadcasted_iota(jnp.int32, (R, C), 0) & s
    return jnp.where(bit != 0, pltpu.roll(x, s, 0), pltpu.roll(x, R - s, 0))


def _sort_kernel(noise_ref, shuf_ref, *, L):
    ir = jax.lax.broadcasted_iota(jnp.int32, (R, C), 0)
    ic = jax.lax.broadcasted_iota(jnp.int32, (R, C), 1)
    idx = ir * C + ic
    m = (noise_ref[0, :, :] * float(1 << 23)).astype(jnp.int32)

    def bit_of(v):
        if v < C:
            return (ic & v) != 0
        return (ir & (v // C)) != 0

    k = 2
    while k <= L:
        d = k // 2
        while d >= 1:
            pm = _xor_shuffle(m, d)
            pidx = _xor_shuffle(idx, d)
            p_lt = (pm < m) | ((pm == m) & (pidx < idx))
            want_min = bit_of(k) == bit_of(d)
            take = p_lt == want_min
            m = jnp.where(take, pm, m)
            idx = jnp.where(take, pidx, idx)
            d //= 2
        k *= 2
    shuf_ref[0, :, :] = idx


def _restore_kernel(shrow_ref, shcol_ref, rest_ref, mask_ref, *, L, len_keep):
    sh_row = shrow_ref[0, :, :]
    sh_col = shcol_ref[0, :, :]
    ihi = jax.lax.broadcasted_iota(jnp.int32, (R, 1), 0)
    ilo = jax.lax.broadcasted_iota(jnp.int32, (1, C), 1)
    a = ((sh_row >> 7) == ihi).astype(jnp.float32)
    kcol = jax.lax.broadcasted_iota(jnp.int32, (L, 1), 0)
    b = jnp.where((sh_col & (C - 1)) == ilo, kcol, 0).astype(jnp.float32)
    mres = jnp.dot(a, b, precision=jax.lax.Precision.HIGHEST)
    rest = mres.astype(jnp.int32)
    rest_ref[0, :, :] = rest
    mask_ref[0, :, :] = (rest >= len_keep).astype(jnp.float32)


def _bcast_kernel(col_ref, out_ref, *, E, rows):
    out_ref[0, :, :] = jnp.broadcast_to(col_ref[0, :, :], (rows, E))


def kernel(x):
    B, L, E = x.shape
    len_keep = int(L * (1.0 - MASK_RATIO_))
    noise = jax.random.uniform(
        jax.random.fold_in(jax.random.key(0), 1), (B, L), dtype=jnp.float32)
    noise_g = noise.reshape(B, R, C)

    shuf = pl.pallas_call(
        functools.partial(_sort_kernel, L=L),
        grid=(B,),
        in_specs=[pl.BlockSpec((1, R, C), lambda b: (b, 0, 0))],
        out_specs=pl.BlockSpec((1, R, C), lambda b: (b, 0, 0)),
        out_shape=jax.ShapeDtypeStruct((B, R, C), jnp.int32),
    )(noise_g)

    sh_row = shuf.reshape(B, 1, L)
    sh_col = shuf.reshape(B, L, 1)

    rest, mask = pl.pallas_call(
        functools.partial(_restore_kernel, L=L, len_keep=len_keep),
        grid=(B,),
        in_specs=[
            pl.BlockSpec((1, 1, L), lambda b: (b, 0, 0)),
            pl.BlockSpec((1, L, 1), lambda b: (b, 0, 0)),
        ],
        out_specs=[
            pl.BlockSpec((1, R, C), lambda b: (b, 0, 0)),
            pl.BlockSpec((1, R, C), lambda b: (b, 0, 0)),
        ],
        out_shape=[
            jax.ShapeDtypeStruct((B, R, C), jnp.int32),
            jax.ShapeDtypeStruct((B, R, C), jnp.float32),
        ],
    )(sh_row, sh_col)

    def bcast(nrows, off):
        rows = min(KTILE, nrows)
        return pl.pallas_call(
            functools.partial(_bcast_kernel, E=E, rows=rows),
            grid=(B, nrows // rows),
            in_specs=[pl.BlockSpec((1, rows, 1), lambda b, k: (b, k + off, 0))],
            out_specs=pl.BlockSpec((1, rows, E), lambda b, k: (b, k, 0)),
            out_shape=jax.ShapeDtypeStruct((B, nrows, E), jnp.int32),
        )(sh_col)

    ids_keep = bcast(len_keep, 0)
    ids_mask = bcast(L - len_keep, len_keep // KTILE)

    return (mask.reshape(B, L), ids_keep, rest.reshape(B, L), ids_mask)


# speedup vs baseline: 5.3658x; 1.0051x over previous
import functools

import jax
import jax.numpy as jnp
from jax.experimental import pallas as pl
from jax.experimental.pallas import tpu as pltpu

MASK_RATIO_ = 0.75
R, C = 64, 128
KTILE = 2048


def _xor_shuffle(x, d):
    if d < C:
        bit = jax.lax.broadcasted_iota(jnp.int32, (R, C), 1) & d
        return jnp.where(bit != 0, pltpu.roll(x, d, 1), pltpu.roll(x, C - d, 1))
    s = d // C
    bit = jax.lax.broadcasted_iota(jnp.int32, (R, C), 0) & s
    return jnp.where(bit != 0, pltpu.roll(x, s, 0), pltpu.roll(x, R - s, 0))


def _sort_kernel(noise_ref, shuf_ref, *, L):
    ir = jax.lax.broadcasted_iota(jnp.int32, (R, C), 0)
    ic = jax.lax.broadcasted_iota(jnp.int32, (R, C), 1)
    idx = ir * C + ic
    m = (noise_ref[0, :, :] * float(1 << 23)).astype(jnp.int32)

    def bit_of(v):
        if v < C:
            return (ic & v) != 0
        return (ir & (v // C)) != 0

    k = 2
    while k <= L:
        d = k // 2
        while d >= 1:
            pm = _xor_shuffle(m, d)
            pidx = _xor_shuffle(idx, d)
            p_lt = (pm < m) | ((pm == m) & (pidx < idx))
            want_min = bit_of(k) == bit_of(d)
            take = p_lt == want_min
            m = jnp.where(take, pm, m)
            idx = jnp.where(take, pidx, idx)
            d //= 2
        k *= 2
    shuf_ref[0, :, :] = idx


def _restore_kernel(shrow_ref, shcol_ref, rest_ref, mask_ref, *, L, len_keep):
    sh_row = shrow_ref[0, :, :]
    sh_col = shcol_ref[0, :, :]
    ihi = jax.lax.broadcasted_iota(jnp.int32, (R, 1), 0)
    ilo = jax.lax.broadcasted_iota(jnp.int32, (1, C), 1)
    a = ((sh_row >> 7) == ihi).astype(jnp.float32)
    kcol = jax.lax.broadcasted_iota(jnp.int32, (L, 1), 0)
    b = jnp.where((sh_col & (C - 1)) == ilo, kcol, 0).astype(jnp.float32)
    mres = jnp.dot(a, b, precision=jax.lax.Precision.HIGHEST)
    rest = mres.astype(jnp.int32)
    rest_ref[0, :, :] = rest
    mask_ref[0, :, :] = jnp.where(mres >= float(len_keep), 1.0, 0.0)


def _bcast_kernel(col_ref, out_ref, *, E, rows):
    out_ref[0, :, :] = jnp.broadcast_to(col_ref[0, :, :], (rows, E))


def kernel(x):
    B, L, E = x.shape
    len_keep = int(L * (1.0 - MASK_RATIO_))
    noise = jax.random.uniform(
        jax.random.fold_in(jax.random.key(0), 1), (B, L), dtype=jnp.float32)
    noise_g = noise.reshape(B, R, C)

    shuf = pl.pallas_call(
        functools.partial(_sort_kernel, L=L),
        grid=(B,),
        in_specs=[pl.BlockSpec((1, R, C), lambda b: (b, 0, 0))],
        out_specs=pl.BlockSpec((1, R, C), lambda b: (b, 0, 0)),
        out_shape=jax.ShapeDtypeStruct((B, R, C), jnp.int32),
    )(noise_g)

    sh_row = shuf.reshape(B, 1, L)
    sh_col = shuf.reshape(B, L, 1)

    rest, mask = pl.pallas_call(
        functools.partial(_restore_kernel, L=L, len_keep=len_keep),
        grid=(B,),
        in_specs=[
            pl.BlockSpec((1, 1, L), lambda b: (b, 0, 0)),
            pl.BlockSpec((1, L, 1), lambda b: (b, 0, 0)),
        ],
        out_specs=[
            pl.BlockSpec((1, R, C), lambda b: (b, 0, 0)),
            pl.BlockSpec((1, R, C), lambda b: (b, 0, 0)),
        ],
        out_shape=[
            jax.ShapeDtypeStruct((B, R, C), jnp.int32),
            jax.ShapeDtypeStruct((B, R, C), jnp.float32),
        ],
    )(sh_row, sh_col)

    def bcast(nrows, off):
        rows = min(KTILE, nrows)
        return pl.pallas_call(
            functools.partial(_bcast_kernel, E=E, rows=rows),
            grid=(B, nrows // rows),
            in_specs=[pl.BlockSpec((1, rows, 1), lambda b, k: (b, k + off, 0))],
            out_specs=pl.BlockSpec((1, rows, E), lambda b, k: (b, k, 0)),
            out_shape=jax.ShapeDtypeStruct((B, nrows, E), jnp.int32),
        )(sh_col)

    ids_keep = bcast(len_keep, 0)
    ids_mask = bcast(L - len_keep, len_keep // KTILE)

    return (mask.reshape(B, L), ids_keep, rest.reshape(B, L), ids_mask)
